# baseline (speedup 1.0000x reference)
import functools
import math

import jax
import jax.numpy as jnp
from jax import lax
from jax.experimental import pallas as pl
from jax.experimental.pallas import tpu as pltpu
from jax.experimental.pallas import tpu_sc as plsc

B, S, D_MODEL, D_FF, E, TOPK = 1, 2048, 768, 2048, 8, 2
T = B * S
TM = 128
NTILES = (TOPK * T) // TM + E
NPAD = NTILES * TM

NC, NS = 2, 16
NW = NC * NS
TPW = T // NW
SUB = 32

_SQRT2 = math.sqrt(2.0)


def _gelu_exact(v):
    return 0.5 * v * (1.0 + jax.lax.erf(v / _SQRT2))


def _router_body(x_ref, gw_ref, gb_ref, pos_ref, w0_ref, w1_ref, meta_ref):
    xt = x_ref[...]
    logits = jnp.dot(xt, gw_ref[...], preferred_element_type=jnp.float32)
    logits = logits + gb_ref[...]
    lane = lax.broadcasted_iota(jnp.int32, (T, E), 1)
    m1 = jnp.max(logits, axis=-1, keepdims=True)
    am1 = jnp.min(jnp.where(logits == m1, lane, E), axis=-1, keepdims=True)
    l2 = jnp.where(lane == am1, -jnp.inf, logits)
    m2 = jnp.max(l2, axis=-1, keepdims=True)
    am2 = jnp.min(jnp.where(l2 == m2, lane, E), axis=-1, keepdims=True)
    p1 = 1.0 / (1.0 + jnp.exp(m2 - m1))
    p2 = 1.0 - p1

    oh0 = jnp.where(lane == am1, 1.0, 0.0)
    oh1 = jnp.where(lane == am2, 1.0, 0.0)
    oh = jnp.concatenate([oh0, oh1], axis=1)
    r_io = lax.broadcasted_iota(jnp.int32, (T, T), 0)
    c_io = lax.broadcasted_iota(jnp.int32, (T, T), 1)
    tri = jnp.where(r_io >= c_io, 1.0, 0.0)
    inc = jnp.dot(tri, oh, preferred_element_type=jnp.float32)
    inc0, inc1 = inc[:, :E], inc[:, E:]
    tot0 = inc0[T - 1:T, :]
    tot1 = inc1[T - 1:T, :]
    count = tot0 + tot1

    tiles = jnp.floor((count + (TM - 1)) * (1.0 / TM))
    r8 = lax.broadcasted_iota(jnp.int32, (E, E), 0)
    c8 = lax.broadcasted_iota(jnp.int32, (E, E), 1)
    ut8 = jnp.where(r8 <= c8, 1.0, 0.0)
    cumtiles = jnp.dot(tiles, ut8, preferred_element_type=jnp.float32)
    offp = (cumtiles - tiles) * TM
    total_tiles = jnp.max(cumtiles)

    pos0 = jnp.sum(oh0 * (inc0 + offp), axis=1, keepdims=True) - 1.0
    pos1 = jnp.sum(oh1 * (inc1 + offp + tot0), axis=1, keepdims=True) - 1.0
    pos_ref[...] = jnp.concatenate([pos0, pos1], axis=1).astype(jnp.int32)

    ones16 = jnp.ones((1, 16), jnp.float32)
    w0_ref[...] = p1 * ones16
    w1_ref[...] = p2 * ones16

    st = jnp.reshape(cumtiles - tiles, (E, 1))
    nt = jnp.reshape(tiles, (E, 1))
    meta_ref[...] = jnp.concatenate([st, nt], axis=1).astype(jnp.int32)


def _router(xf, gate_w, gate_b):
    return pl.pallas_call(
        _router_body,
        in_specs=[
            pl.BlockSpec((T, D_MODEL), lambda: (0, 0)),
            pl.BlockSpec((D_MODEL, E), lambda: (0, 0)),
            pl.BlockSpec((1, E), lambda: (0, 0)),
        ],
        out_specs=[
            pl.BlockSpec((T, TOPK), lambda: (0, 0)),
            pl.BlockSpec((T, 16), lambda: (0, 0)),
            pl.BlockSpec((T, 16), lambda: (0, 0)),
            pl.BlockSpec((E, 2), lambda: (0, 0)),
        ],
        out_shape=[
            jax.ShapeDtypeStruct((T, TOPK), jnp.int32),
            jax.ShapeDtypeStruct((T, 16), jnp.float32),
            jax.ShapeDtypeStruct((T, 16), jnp.float32),
            jax.ShapeDtypeStruct((E, 2), jnp.int32),
        ],
    )(xf, gate_w, gate_b.reshape(1, E))


@functools.cache
def _sc_mesh():
    return plsc.VectorSubcoreMesh(core_axis_name="c", subcore_axis_name="s",
                                  num_cores=NC, num_subcores=NS)


@functools.cache
def _dispatch_kernel():
    @functools.partial(
        pl.kernel,
        out_type=jax.ShapeDtypeStruct((NPAD, D_MODEL), jnp.float32),
        mesh=_sc_mesh(),
        scratch_types=[
            pltpu.VMEM((TPW, D_MODEL), jnp.float32),
            pltpu.VMEM((TPW,), jnp.int32),
            pltpu.VMEM((TPW,), jnp.int32),
            pltpu.SemaphoreType.DMA,
        ],
    )
    def _dispatch(x_hbm, pos_hbm, xg_hbm, xbuf, idx0, idx1, sem):
        wid = lax.axis_index("s") * NC + lax.axis_index("c")
        base = wid * TPW
        pltpu.sync_copy(x_hbm.at[pl.ds(base, TPW)], xbuf)
        pltpu.sync_copy(pos_hbm.at[0, pl.ds(base, TPW)], idx0)
        pltpu.sync_copy(pos_hbm.at[1, pl.ds(base, TPW)], idx1)
        pltpu.async_copy(xbuf, xg_hbm.at[idx0], sem).wait()
        pltpu.async_copy(xbuf, xg_hbm.at[idx1], sem).wait()

    return _dispatch


def _ffn_body(st_ref, nt_ref, xg_ref, w1_ref, b1_ref, w2_ref, b2_ref,
              y_hbm, yt0, yt1, sem0, sem1):
    e = pl.program_id(0)
    st = st_ref[e]
    nt = nt_ref[e]

    def tile(k, carry):
        t = st + k
        xtile = xg_ref[pl.ds(t * TM, TM), :]
        h = _gelu_exact(
            jnp.dot(xtile, w1_ref[0], preferred_element_type=jnp.float32)
            + b1_ref[0])
        yv = (jnp.dot(h, w2_ref[0], preferred_element_type=jnp.float32)
              + b2_ref[0])

        @pl.when(lax.rem(k, 2) == 0)
        def _():
            @pl.when(k >= 2)
            def _():
                pltpu.make_async_copy(
                    yt0, y_hbm.at[pl.ds((t - 2) * TM, TM), :], sem0).wait()

            yt0[...] = yv
            pltpu.make_async_copy(
                yt0, y_hbm.at[pl.ds(t * TM, TM), :], sem0).start()

        @pl.when(lax.rem(k, 2) == 1)
        def _():
            @pl.when(k >= 3)
            def _():
                pltpu.make_async_copy(
                    yt1, y_hbm.at[pl.ds((t - 2) * TM, TM), :], sem1).wait()

            yt1[...] = yv
            pltpu.make_async_copy(
                yt1, y_hbm.at[pl.ds(t * TM, TM), :], sem1).start()

        return carry

    lax.fori_loop(0, nt, tile, 0)

    @pl.when(nt >= 1)
    def _():
        sem = lax.rem(nt - 1, 2)

        @pl.when(sem == 0)
        def _():
            pltpu.make_async_copy(
                yt0, y_hbm.at[pl.ds((st + nt - 1) * TM, TM), :], sem0).wait()

        @pl.when(sem == 1)
        def _():
            pltpu.make_async_copy(
                yt1, y_hbm.at[pl.ds((st + nt - 1) * TM, TM), :], sem1).wait()

    @pl.when(nt >= 2)
    def _():
        sem = lax.rem(nt - 2, 2)

        @pl.when(sem == 0)
        def _():
            pltpu.make_async_copy(
                yt0, y_hbm.at[pl.ds((st + nt - 2) * TM, TM), :], sem0).wait()

        @pl.when(sem == 1)
        def _():
            pltpu.make_async_copy(
                yt1, y_hbm.at[pl.ds((st + nt - 2) * TM, TM), :], sem1).wait()


def _ffn(xg, W1, b1, W2, b2, st, nt):
    return pl.pallas_call(
        _ffn_body,
        grid_spec=pltpu.PrefetchScalarGridSpec(
            num_scalar_prefetch=2,
            grid=(E,),
            in_specs=[
                pl.BlockSpec((NPAD, D_MODEL), lambda e, st, nt: (0, 0)),
                pl.BlockSpec((1, D_MODEL, D_FF), lambda e, st, nt: (e, 0, 0)),
                pl.BlockSpec((1, 1, D_FF), lambda e, st, nt: (e, 0, 0)),
                pl.BlockSpec((1, D_FF, D_MODEL), lambda e, st, nt: (e, 0, 0)),
                pl.BlockSpec((1, 1, D_MODEL), lambda e, st, nt: (e, 0, 0)),
            ],
            out_specs=pl.BlockSpec(memory_space=pl.ANY),
            scratch_shapes=[
                pltpu.VMEM((TM, D_MODEL), jnp.float32),
                pltpu.VMEM((TM, D_MODEL), jnp.float32),
                pltpu.SemaphoreType.DMA,
                pltpu.SemaphoreType.DMA,
            ],
        ),
        out_shape=jax.ShapeDtypeStruct((NPAD, D_MODEL), jnp.float32),
    )(st, nt, xg, W1, b1.reshape(E, 1, D_FF), W2, b2.reshape(E, 1, D_MODEL))


@functools.cache
def _combine_kernel():
    @functools.partial(
        pl.kernel,
        out_type=jax.ShapeDtypeStruct((T, D_MODEL), jnp.float32),
        mesh=_sc_mesh(),
        scratch_types=[
            pltpu.VMEM((SUB, D_MODEL), jnp.float32),
            pltpu.VMEM((SUB, D_MODEL), jnp.float32),
            pltpu.VMEM((SUB, D_MODEL), jnp.float32),
            pltpu.VMEM((SUB,), jnp.int32),
            pltpu.VMEM((SUB,), jnp.int32),
            pltpu.VMEM((SUB, 16), jnp.float32),
            pltpu.VMEM((SUB, 16), jnp.float32),
            pltpu.SemaphoreType.DMA,
        ],
    )
    def _combine(y_hbm, pos_hbm, ws_hbm, out_hbm,
                 ya, yb, ob, idx0, idx1, wb0, wb1, sem):
        wid = lax.axis_index("s") * NC + lax.axis_index("c")
        for sub in range(TPW // SUB):
            base = wid * TPW + sub * SUB
            pltpu.sync_copy(pos_hbm.at[0, pl.ds(base, SUB)], idx0)
            pltpu.sync_copy(pos_hbm.at[1, pl.ds(base, SUB)], idx1)
            pltpu.sync_copy(ws_hbm.at[0, pl.ds(base, SUB)], wb0)
            pltpu.sync_copy(ws_hbm.at[1, pl.ds(base, SUB)], wb1)
            ca = pltpu.async_copy(y_hbm.at[idx0], ya, sem)
            cb = pltpu.async_copy(y_hbm.at[idx1], yb, sem)
            ca.wait()
            cb.wait()

            @plsc.parallel_loop(0, SUB, 1)
            def _row(r):
                w0 = wb0[r]
                w1 = wb1[r]

                @plsc.parallel_loop(0, D_MODEL // 16, 1, unroll=4)
                def _col(c):
                    sl = pl.ds(c * 16, 16)
                    ob[r, sl] = w0 * ya[r, sl] + w1 * yb[r, sl]

            pltpu.sync_copy(ob, out_hbm.at[pl.ds(base, SUB)])

    return _combine


@jax.jit
def _moe(x, gate_w, gate_b, W1, b1, W2, b2):
    xf = x.reshape(T, D_MODEL)
    pos_tk, w016, w116, meta = _router(xf, gate_w, gate_b)
    pos = pos_tk.T
    ws = jnp.stack([w016, w116])
    xg = _dispatch_kernel()(xf, pos)
    y = _ffn(xg, W1, b1, W2, b2, meta[:, 0], meta[:, 1])
    out = _combine_kernel()(y, pos, ws)
    return out.reshape(B, S, D_MODEL)


def kernel(x, gate_w, gate_b, W1, b1, W2, b2):
    return _moe(x, gate_w, gate_b, W1, b1, W2, b2)


# speedup vs baseline: 1.0380x; 1.0380x over previous
import functools
import math

import jax
import jax.numpy as jnp
from jax import lax
from jax.experimental import pallas as pl
from jax.experimental.pallas import tpu as pltpu
from jax.experimental.pallas import tpu_sc as plsc

B, S, D_MODEL, D_FF, E, TOPK = 1, 2048, 768, 2048, 8, 2
T = B * S
TM = 256
NTILES = (TOPK * T) // TM + E
NPAD = NTILES * TM

NC, NS = 2, 16
NW = NC * NS
TPW = T // NW
CSUB = 16

_SQRT2 = math.sqrt(2.0)


def _gelu_exact(v):
    return 0.5 * v * (1.0 + jax.lax.erf(v / _SQRT2))


def _router_body(x_ref, gw_ref, gb_ref, pos_ref, w0_ref, w1_ref, meta_ref):
    xt = x_ref[...]
    logits = jnp.dot(xt, gw_ref[...], preferred_element_type=jnp.float32)
    logits = logits + gb_ref[...]
    lane = lax.broadcasted_iota(jnp.int32, (T, E), 1)
    m1 = jnp.max(logits, axis=-1, keepdims=True)
    am1 = jnp.min(jnp.where(logits == m1, lane, E), axis=-1, keepdims=True)
    l2 = jnp.where(lane == am1, -jnp.inf, logits)
    m2 = jnp.max(l2, axis=-1, keepdims=True)
    am2 = jnp.min(jnp.where(l2 == m2, lane, E), axis=-1, keepdims=True)
    p1 = 1.0 / (1.0 + jnp.exp(m2 - m1))
    p2 = 1.0 - p1

    oh0 = jnp.where(lane == am1, 1.0, 0.0)
    oh1 = jnp.where(lane == am2, 1.0, 0.0)
    oh = jnp.concatenate([oh0, oh1], axis=1)
    r_io = lax.broadcasted_iota(jnp.int32, (T, T), 0)
    c_io = lax.broadcasted_iota(jnp.int32, (T, T), 1)
    tri = jnp.where(r_io >= c_io, 1.0, 0.0)
    inc = jnp.dot(tri, oh, preferred_element_type=jnp.float32)
    inc0, inc1 = inc[:, :E], inc[:, E:]
    tot0 = inc0[T - 1:T, :]
    tot1 = inc1[T - 1:T, :]
    count = tot0 + tot1

    tiles = jnp.floor((count + (TM - 1)) * (1.0 / TM))
    r8 = lax.broadcasted_iota(jnp.int32, (E, E), 0)
    c8 = lax.broadcasted_iota(jnp.int32, (E, E), 1)
    ut8 = jnp.where(r8 <= c8, 1.0, 0.0)
    cumtiles = jnp.dot(tiles, ut8, preferred_element_type=jnp.float32)
    offp = (cumtiles - tiles) * TM
    total_tiles = jnp.max(cumtiles)

    pos0 = jnp.sum(oh0 * (inc0 + offp), axis=1, keepdims=True) - 1.0
    pos1 = jnp.sum(oh1 * (inc1 + offp + tot0), axis=1, keepdims=True) - 1.0
    pos_ref[...] = jnp.concatenate([pos0, pos1], axis=1).astype(jnp.int32)

    ones16 = jnp.ones((1, 16), jnp.float32)
    w0_ref[...] = p1 * ones16
    w1_ref[...] = p2 * ones16

    st = jnp.reshape(cumtiles - tiles, (E, 1))
    nt = jnp.reshape(tiles, (E, 1))
    meta_ref[...] = jnp.concatenate([st, nt], axis=1).astype(jnp.int32)


def _router(xf, gate_w, gate_b):
    return pl.pallas_call(
        _router_body,
        in_specs=[
            pl.BlockSpec((T, D_MODEL), lambda: (0, 0)),
            pl.BlockSpec((D_MODEL, E), lambda: (0, 0)),
            pl.BlockSpec((1, E), lambda: (0, 0)),
        ],
        out_specs=[
            pl.BlockSpec((T, TOPK), lambda: (0, 0)),
            pl.BlockSpec((T, 16), lambda: (0, 0)),
            pl.BlockSpec((T, 16), lambda: (0, 0)),
            pl.BlockSpec((E, 2), lambda: (0, 0)),
        ],
        out_shape=[
            jax.ShapeDtypeStruct((T, TOPK), jnp.int32),
            jax.ShapeDtypeStruct((T, 16), jnp.float32),
            jax.ShapeDtypeStruct((T, 16), jnp.float32),
            jax.ShapeDtypeStruct((E, 2), jnp.int32),
        ],
    )(xf, gate_w, gate_b.reshape(1, E))


@functools.cache
def _sc_mesh():
    return plsc.VectorSubcoreMesh(core_axis_name="c", subcore_axis_name="s",
                                  num_cores=NC, num_subcores=NS)


@functools.cache
def _dispatch_kernel():
    @functools.partial(
        pl.kernel,
        out_type=jax.ShapeDtypeStruct((NPAD, D_MODEL), jnp.float32),
        mesh=_sc_mesh(),
        scratch_types=[
            pltpu.VMEM((TPW, D_MODEL), jnp.float32),
            pltpu.VMEM((TPW,), jnp.int32),
            pltpu.VMEM((TPW,), jnp.int32),
            pltpu.SemaphoreType.DMA,
        ],
    )
    def _dispatch(x_hbm, pos_hbm, xg_hbm, xbuf, idx0, idx1, sem):
        wid = lax.axis_index("s") * NC + lax.axis_index("c")
        base = wid * TPW
        pltpu.sync_copy(x_hbm.at[pl.ds(base, TPW)], xbuf)
        pltpu.sync_copy(pos_hbm.at[0, pl.ds(base, TPW)], idx0)
        pltpu.sync_copy(pos_hbm.at[1, pl.ds(base, TPW)], idx1)
        pltpu.async_copy(xbuf, xg_hbm.at[idx0], sem).wait()
        pltpu.async_copy(xbuf, xg_hbm.at[idx1], sem).wait()

    return _dispatch


def _ffn_body(st_ref, nt_ref, xg_ref, w1_ref, b1_ref, w2_ref, b2_ref,
              y_hbm, yt0, yt1, sem0, sem1):
    e = pl.program_id(0)
    st = st_ref[e]
    nt = nt_ref[e]

    def tile(k, carry):
        t = st + k
        xtile = xg_ref[pl.ds(t * TM, TM), :]
        h = _gelu_exact(
            jnp.dot(xtile, w1_ref[0], preferred_element_type=jnp.float32)
            + b1_ref[0])
        yv = (jnp.dot(h, w2_ref[0], preferred_element_type=jnp.float32)
              + b2_ref[0])

        @pl.when(lax.rem(k, 2) == 0)
        def _():
            @pl.when(k >= 2)
            def _():
                pltpu.make_async_copy(
                    yt0, y_hbm.at[pl.ds((t - 2) * TM, TM), :], sem0).wait()

            yt0[...] = yv
            pltpu.make_async_copy(
                yt0, y_hbm.at[pl.ds(t * TM, TM), :], sem0).start()

        @pl.when(lax.rem(k, 2) == 1)
        def _():
            @pl.when(k >= 3)
            def _():
                pltpu.make_async_copy(
                    yt1, y_hbm.at[pl.ds((t - 2) * TM, TM), :], sem1).wait()

            yt1[...] = yv
            pltpu.make_async_copy(
                yt1, y_hbm.at[pl.ds(t * TM, TM), :], sem1).start()

        return carry

    lax.fori_loop(0, nt, tile, 0)

    @pl.when(nt >= 1)
    def _():
        sem = lax.rem(nt - 1, 2)

        @pl.when(sem == 0)
        def _():
            pltpu.make_async_copy(
                yt0, y_hbm.at[pl.ds((st + nt - 1) * TM, TM), :], sem0).wait()

        @pl.when(sem == 1)
        def _():
            pltpu.make_async_copy(
                yt1, y_hbm.at[pl.ds((st + nt - 1) * TM, TM), :], sem1).wait()

    @pl.when(nt >= 2)
    def _():
        sem = lax.rem(nt - 2, 2)

        @pl.when(sem == 0)
        def _():
            pltpu.make_async_copy(
                yt0, y_hbm.at[pl.ds((st + nt - 2) * TM, TM), :], sem0).wait()

        @pl.when(sem == 1)
        def _():
            pltpu.make_async_copy(
                yt1, y_hbm.at[pl.ds((st + nt - 2) * TM, TM), :], sem1).wait()


def _ffn(xg, W1, b1, W2, b2, st, nt):
    return pl.pallas_call(
        _ffn_body,
        grid_spec=pltpu.PrefetchScalarGridSpec(
            num_scalar_prefetch=2,
            grid=(E,),
            in_specs=[
                pl.BlockSpec((NPAD, D_MODEL), lambda e, st, nt: (0, 0)),
                pl.BlockSpec((1, D_MODEL, D_FF), lambda e, st, nt: (e, 0, 0)),
                pl.BlockSpec((1, 1, D_FF), lambda e, st, nt: (e, 0, 0)),
                pl.BlockSpec((1, D_FF, D_MODEL), lambda e, st, nt: (e, 0, 0)),
                pl.BlockSpec((1, 1, D_MODEL), lambda e, st, nt: (e, 0, 0)),
            ],
            out_specs=pl.BlockSpec(memory_space=pl.ANY),
            scratch_shapes=[
                pltpu.VMEM((TM, D_MODEL), jnp.float32),
                pltpu.VMEM((TM, D_MODEL), jnp.float32),
                pltpu.SemaphoreType.DMA,
                pltpu.SemaphoreType.DMA,
            ],
        ),
        out_shape=jax.ShapeDtypeStruct((NPAD, D_MODEL), jnp.float32),
    )(st, nt, xg, W1, b1.reshape(E, 1, D_FF), W2, b2.reshape(E, 1, D_MODEL))


@functools.cache
def _combine_kernel():
    nsub = TPW // CSUB

    @functools.partial(
        pl.kernel,
        out_type=jax.ShapeDtypeStruct((T, D_MODEL), jnp.float32),
        mesh=_sc_mesh(),
        scratch_types=(
            [pltpu.VMEM((CSUB, D_MODEL), jnp.float32)] * 4
            + [pltpu.VMEM((CSUB, D_MODEL), jnp.float32)] * 2
            + [pltpu.VMEM((CSUB,), jnp.int32)] * 4
            + [pltpu.VMEM((CSUB, 16), jnp.float32)] * 4
            + [pltpu.SemaphoreType.DMA] * 2
        ),
    )
    def _combine(y_hbm, pos_hbm, ws_hbm, out_hbm,
                 ya0, yb0, ya1, yb1, ob0, ob1,
                 i00, i10, i01, i11, w00, w10, w01, w11, sem0, sem1):
        wid = lax.axis_index("s") * NC + lax.axis_index("c")
        yas, ybs = [ya0, ya1], [yb0, yb1]
        obs = [ob0, ob1]
        i0s, i1s = [i00, i01], [i10, i11]
        w0s, w1s = [w00, w01], [w10, w11]
        sems = [sem0, sem1]
        handles = [None, None]

        def issue(sub):
            s = sub % 2
            base = wid * TPW + sub * CSUB
            pltpu.sync_copy(pos_hbm.at[0, pl.ds(base, CSUB)], i0s[s])
            pltpu.sync_copy(pos_hbm.at[1, pl.ds(base, CSUB)], i1s[s])
            pltpu.sync_copy(ws_hbm.at[0, pl.ds(base, CSUB)], w0s[s])
            pltpu.sync_copy(ws_hbm.at[1, pl.ds(base, CSUB)], w1s[s])
            ca = pltpu.async_copy(y_hbm.at[i0s[s]], yas[s], sems[s])
            cb = pltpu.async_copy(y_hbm.at[i1s[s]], ybs[s], sems[s])
            handles[s] = (ca, cb)

        def process(sub):
            s = sub % 2
            ca, cb = handles[s]
            ca.wait()
            cb.wait()
            ya, yb, ob, wb0, wb1 = yas[s], ybs[s], obs[s], w0s[s], w1s[s]

            @plsc.parallel_loop(0, CSUB, 1)
            def _row(r):
                w0 = wb0[r]
                w1 = wb1[r]

                @plsc.parallel_loop(0, D_MODEL // 16, 1, unroll=4)
                def _col(c):
                    sl = pl.ds(c * 16, 16)
                    ob[r, sl] = w0 * ya[r, sl] + w1 * yb[r, sl]

            base = wid * TPW + sub * CSUB
            pltpu.sync_copy(ob, out_hbm.at[pl.ds(base, CSUB)])

        issue(0)
        for sub in range(nsub):
            if sub + 1 < nsub:
                issue(sub + 1)
            process(sub)

    return _combine


@jax.jit
def _moe(x, gate_w, gate_b, W1, b1, W2, b2):
    xf = x.reshape(T, D_MODEL)
    pos_tk, w016, w116, meta = _router(xf, gate_w, gate_b)
    pos = pos_tk.T
    ws = jnp.stack([w016, w116])
    xg = _dispatch_kernel()(xf, pos)
    y = _ffn(xg, W1, b1, W2, b2, meta[:, 0], meta[:, 1])
    out = _combine_kernel()(y, pos, ws)
    return out.reshape(B, S, D_MODEL)


def kernel(x, gate_w, gate_b, W1, b1, W2, b2):
    return _moe(x, gate_w, gate_b, W1, b1, W2, b2)
